# baseline (speedup 1.0000x reference)
import functools

import jax
import jax.numpy as jnp
from jax import lax
from jax.experimental import pallas as pl
from jax.experimental.pallas import tpu as pltpu
from jax.experimental.pallas import tpu_sc as plsc

N = 10000
E = 320000
N_GRAPHS = 64

NC = 2
NS = 16
NW = NC * NS
CHUNK = 100
NCHUNK_W = E // (NW * CHUNK)
NCHUNK_T = E // (NS * CHUNK)
STRIPE = 640
NP = NS * STRIPE


def _make_sc_agg(F):
    Fh = F // 2
    mesh = plsc.VectorSubcoreMesh(core_axis_name="c", subcore_axis_name="s")

    @functools.partial(
        pl.kernel, mesh=mesh,
        compiler_params=pltpu.CompilerParams(use_tc_tiling_on_sc=False),
        out_type=jax.ShapeDtypeStruct((NC, NP, Fh), jnp.float32),
        scratch_types=[
            pltpu.VMEM((NCHUNK_T, CHUNK), jnp.int32),
            pltpu.VMEM((NCHUNK_T, CHUNK), jnp.int32),
            [pltpu.VMEM((CHUNK, Fh), jnp.float32)] * 4,
            pltpu.VMEM((64, Fh), jnp.float32),
            pltpu.VMEM_SHARED((NP, Fh), jnp.float32),
            pltpu.VMEM_SHARED((N, Fh), jnp.float32),
            [pltpu.SemaphoreType.DMA] * 4,
            [pltpu.SemaphoreType.DMA] * 4,
        ],
    )
    def k(g_hbm, e4, out_hbm, src_v, dst_v, bufs, zbuf, acc, g_sp, sems,
          ssems):
        c = lax.axis_index("c")
        s = lax.axis_index("s")

        for i in range(64):
            for j in range(Fh // 16):
                zbuf[i, 16 * j:16 * (j + 1)] = jnp.zeros((16,), jnp.float32)

        def zstripe(kk, carry):
            pltpu.sync_copy(zbuf, acc.at[pl.ds(s * STRIPE + kk * 64, 64)])
            return carry
        lax.fori_loop(0, STRIPE // 64, zstripe, 0)
        pltpu.sync_copy(g_hbm.at[c].at[pl.ds(s * (N // NS), N // NS)],
                        g_sp.at[pl.ds(s * (N // NS), N // NS)])
        plsc.subcore_barrier()

        pltpu.sync_copy(e4.at[0].at[s], src_v)
        pltpu.sync_copy(e4.at[1].at[s], dst_v)

        def gath(j, b):
            pltpu.async_copy(g_sp.at[src_v.at[j]], bufs[b], sems[b])

        def wait(j, b):
            pltpu.make_async_copy(g_sp.at[src_v.at[j]], bufs[b],
                                  sems[b]).wait()

        def scat(j, b):
            pltpu.async_copy(bufs[b], acc.at[dst_v.at[j]], ssems[b],
                             add=True)

        def swait(j, b):
            pltpu.make_async_copy(bufs[b], acc.at[dst_v.at[j]],
                                  ssems[b]).wait()

        for b in range(4):
            gath(b, b)

        def quad(t, carry):
            j0 = 4 * t
            for b in range(4):
                wait(j0 + b, b)
                scat(j0 + b, b)
            for b in range(4):
                swait(j0 + b, b)
                gath(j0 + b + 4, b)
            return carry
        lax.fori_loop(0, NCHUNK_T // 4 - 1, quad, 0)
        for b in range(4):
            j = NCHUNK_T - 4 + b
            wait(j, b)
            scat(j, b)
        for b in range(4):
            swait(NCHUNK_T - 4 + b, b)

        plsc.subcore_barrier()
        pltpu.sync_copy(acc.at[pl.ds(s * STRIPE, STRIPE)],
                        out_hbm.at[c, pl.ds(s * STRIPE, STRIPE)])

    return k


_sc_agg = {f: _make_sc_agg(f) for f in (32, 64)}


def _make_sc_count():
    F = 16
    mesh = plsc.VectorSubcoreMesh(core_axis_name="c", subcore_axis_name="s")

    @functools.partial(
        pl.kernel, mesh=mesh,
        compiler_params=pltpu.CompilerParams(use_tc_tiling_on_sc=False),
        out_type=jax.ShapeDtypeStruct((NC, NP, F), jnp.float32),
        scratch_types=[
            pltpu.VMEM((NCHUNK_W, CHUNK), jnp.int32),
            pltpu.VMEM((CHUNK, F), jnp.float32),
            pltpu.VMEM((64, F), jnp.float32),
            pltpu.VMEM_SHARED((NP, F), jnp.float32),
        ],
    )
    def k(e4, out_hbm, dst_v, ones, zbuf, acc):
        c = lax.axis_index("c")
        s = lax.axis_index("s")

        for i in range(64):
            zbuf[i, 0:16] = jnp.zeros((16,), jnp.float32)
        for i in range(CHUNK):
            ones[i, 0:16] = jnp.ones((16,), jnp.float32)

        def zstripe(kk, carry):
            pltpu.sync_copy(zbuf, acc.at[pl.ds(s * STRIPE + kk * 64, 64)])
            return carry
        lax.fori_loop(0, STRIPE // 64, zstripe, 0)
        plsc.subcore_barrier()

        pltpu.sync_copy(e4.at[1].at[s].at[pl.ds(c * NCHUNK_W, NCHUNK_W)],
                        dst_v)

        def chunk(j, carry):
            pltpu.sync_copy(ones, acc.at[dst_v.at[j]], add=True)
            return carry
        lax.fori_loop(0, NCHUNK_W, chunk, 0)

        plsc.subcore_barrier()
        pltpu.sync_copy(acc.at[pl.ds(s * STRIPE, STRIPE)],
                        out_hbm.at[c, pl.ds(s * STRIPE, STRIPE)])

    return k


_sc_count = _make_sc_count()


def _make_sc_pool():
    F = 64
    mesh = plsc.VectorSubcoreMesh(core_axis_name="c", subcore_axis_name="s")
    NEG = float("-inf")

    @functools.partial(
        pl.kernel, mesh=mesh,
        compiler_params=pltpu.CompilerParams(use_tc_tiling_on_sc=False,
                                             needs_layout_passes=False),
        out_type=jax.ShapeDtypeStruct((N_GRAPHS, F), jnp.float32),
        scratch_types=[
            pltpu.VMEM((N,), jnp.int32),
            pltpu.VMEM((64, F), jnp.float32),
            pltpu.VMEM((2, F), jnp.float32),
        ],
    )
    def k(y_hbm, batch_hbm, out_hbm, b_v, ybuf, obuf):
        c = lax.axis_index("c")
        s = lax.axis_index("s")
        wid = c * NS + s
        g0 = 2 * wid

        pltpu.sync_copy(batch_hbm, b_v)

        def count(i, carry):
            c0, c1, c2 = carry
            one = jnp.ones((16,), jnp.int32)
            zero = jnp.zeros((16,), jnp.int32)
            for u in range(5):
                v = b_v[pl.ds(80 * i + 16 * u, 16)]
                c0 = c0 + jnp.where(v < g0, one, zero)
                c1 = c1 + jnp.where(v < g0 + 1, one, zero)
                c2 = c2 + jnp.where(v < g0 + 2, one, zero)
            return c0, c1, c2
        z = jnp.zeros((16,), jnp.int32)
        c0, c1, c2 = lax.fori_loop(0, N // 80, count, (z, z, z))
        bounds = (jnp.sum(c0), jnp.sum(c1), jnp.sum(c2))

        for gi in range(2):
            lo = bounds[gi]
            hi = bounds[gi + 1]
            n = hi - lo
            minit = tuple(jnp.full((16,), NEG, jnp.float32) for _ in range(4))

            def slab(kk, m):
                pltpu.sync_copy(y_hbm.at[pl.ds(lo + 64 * kk, 64)], ybuf)
                for r in range(64):
                    m = tuple(jnp.maximum(m[q], ybuf[r, 16 * q:16 * (q + 1)])
                              for q in range(4))
                return m
            m = lax.fori_loop(0, n // 64, slab, minit)

            cs = jnp.minimum(jnp.maximum(lo, hi - 64), N - 64)
            pltpu.sync_copy(y_hbm.at[pl.ds(cs, 64)], ybuf)
            for r in range(64):
                ok = jnp.logical_and(cs + r >= lo, cs + r < hi)
                m = tuple(
                    jnp.where(ok,
                              jnp.maximum(m[q], ybuf[r, 16 * q:16 * (q + 1)]),
                              m[q])
                    for q in range(4))

            for q in range(4):
                obuf[gi, 16 * q:16 * (q + 1)] = m[q]

        pltpu.sync_copy(obuf, out_hbm.at[pl.ds(g0, 2)])

    return k


_sc_pool = _make_sc_pool()


def _elu(t):
    return jnp.where(t > 0, t, jnp.exp(t) - 1.0)


GT = 5
BR = N // GT


def _prep_body(cnt_ref, x_ref, w_ref, s_ref, g_ref):
    cnt = cnt_ref[...]
    deg = cnt[0, :, 0:1] + cnt[1, :, 0:1] + 1.0
    sc = jax.lax.rsqrt(deg)
    s_ref[...] = jnp.broadcast_to(sc, s_ref.shape)
    h = jnp.dot(x_ref[...], w_ref[...], preferred_element_type=jnp.float32)
    g = sc * h
    fh = g_ref.shape[2]
    g_ref[0] = g[:, :fh]
    g_ref[1] = g[:, fh:]


def _tc_prep(cnt3, x, W1):
    f = W1.shape[1]
    fh = f // 2
    return pl.pallas_call(
        _prep_body,
        grid=(GT,),
        in_specs=[
            pl.BlockSpec((NC, BR, 16), lambda i: (0, i, 0)),
            pl.BlockSpec((BR, 128), lambda i: (i, 0)),
            pl.BlockSpec((128, f), lambda i: (0, 0)),
        ],
        out_specs=[
            pl.BlockSpec((BR, 8), lambda i: (i, 0)),
            pl.BlockSpec((NC, BR, fh), lambda i: (0, i, 0)),
        ],
        out_shape=[
            jax.ShapeDtypeStruct((NP, 8), jnp.float32),
            jax.ShapeDtypeStruct((NC, N, fh), jnp.float32),
        ],
    )(cnt3, x, W1)


def _layer_body(p_ref, g3_ref, s_ref, w_ref, b_ref, o_ref):
    sc = s_ref[:, 0:1]
    t_lo = sc * (p_ref[0] + g3_ref[0])
    t_hi = sc * (p_ref[1] + g3_ref[1])
    t = jnp.concatenate([t_lo, t_hi], axis=1) + b_ref[...]
    y = _elu(t)
    h = jnp.dot(y, w_ref[...], preferred_element_type=jnp.float32)
    g = sc * h
    fh2 = o_ref.shape[2]
    o_ref[0] = g[:, :fh2]
    o_ref[1] = g[:, fh2:]


def _tc_layer(P3, g3, s, W, b):
    fh = g3.shape[2]
    f2 = W.shape[1]
    fh2 = f2 // 2
    return pl.pallas_call(
        _layer_body,
        grid=(GT,),
        in_specs=[
            pl.BlockSpec((NC, BR, fh), lambda i: (0, i, 0)),
            pl.BlockSpec((NC, BR, fh), lambda i: (0, i, 0)),
            pl.BlockSpec((BR, 8), lambda i: (i, 0)),
            pl.BlockSpec((2 * fh, f2), lambda i: (0, 0)),
            pl.BlockSpec((1, 2 * fh), lambda i: (0, 0)),
        ],
        out_specs=pl.BlockSpec((NC, BR, fh2), lambda i: (0, i, 0)),
        out_shape=jax.ShapeDtypeStruct((NC, N, fh2), jnp.float32),
    )(P3, g3, s, W, b.reshape(1, -1))


def _last_body(p_ref, g3_ref, s_ref, b_ref, o_ref):
    sc = s_ref[:, 0:1]
    t_lo = sc * (p_ref[0] + g3_ref[0])
    t_hi = sc * (p_ref[1] + g3_ref[1])
    t = jnp.concatenate([t_lo, t_hi], axis=1) + b_ref[...]
    o_ref[...] = _elu(t)


def _tc_last(P3, g3, s, b):
    fh = g3.shape[2]
    return pl.pallas_call(
        _last_body,
        grid=(GT,),
        in_specs=[
            pl.BlockSpec((NC, BR, fh), lambda i: (0, i, 0)),
            pl.BlockSpec((NC, BR, fh), lambda i: (0, i, 0)),
            pl.BlockSpec((BR, 8), lambda i: (i, 0)),
            pl.BlockSpec((1, 2 * fh), lambda i: (0, 0)),
        ],
        out_specs=pl.BlockSpec((BR, 2 * fh), lambda i: (i, 0)),
        out_shape=jax.ShapeDtypeStruct((N, 2 * fh), jnp.float32),
    )(P3, g3, s, b.reshape(1, -1))


def _mlp_body(p_ref, w1_ref, b1_ref, w2_ref, b2_ref, w3_ref, b3_ref, o_ref):
    p = p_ref[...]
    p = jnp.where(jnp.isfinite(p), p, 0.0)
    h = _elu(jnp.dot(p, w1_ref[...], preferred_element_type=jnp.float32)
             + b1_ref[...])
    h = _elu(jnp.dot(h, w2_ref[...], preferred_element_type=jnp.float32)
             + b2_ref[...])
    lg = jnp.dot(h, w3_ref[...], preferred_element_type=jnp.float32) \
        + b3_ref[...]
    m = jnp.max(lg, axis=1, keepdims=True)
    lse = m + jnp.log(jnp.sum(jnp.exp(lg - m), axis=1, keepdims=True))
    o_ref[...] = lg - lse


def _tc_mlp(pooled, Wf1, bf1, Wf2, bf2, Wf3, bf3):
    return pl.pallas_call(
        _mlp_body,
        out_shape=jax.ShapeDtypeStruct((N_GRAPHS, 2), jnp.float32),
    )(pooled, Wf1, bf1.reshape(1, -1), Wf2, bf2.reshape(1, -1),
      Wf3, bf3.reshape(1, -1))


def kernel(x, edge_index, batch, W1, b1, W2, b2, Wc0, bc0, Wc1, bc1,
           Wf1, bf1, Wf2, bf2, Wf3, bf3):
    e4 = edge_index.reshape(2, NS, NCHUNK_T, CHUNK)

    cnt3 = _sc_count(e4)
    s, g3 = _tc_prep(cnt3, x, W1)

    def agg(g3_):
        return _sc_agg[2 * g3_.shape[2]](g3_, e4)

    g3 = _tc_layer(agg(g3), g3, s, W2, b1)
    g3 = _tc_layer(agg(g3), g3, s, Wc0, b2)
    g3 = _tc_layer(agg(g3), g3, s, Wc1, bc0)
    y = _tc_last(agg(g3), g3, s, bc1)

    pooled = _sc_pool(y, batch)
    return _tc_mlp(pooled, Wf1, bf1, Wf2, bf2, Wf3, bf3)


# speedup vs baseline: 1.1188x; 1.1188x over previous
import functools

import jax
import jax.numpy as jnp
from jax import lax
from jax.experimental import pallas as pl
from jax.experimental.pallas import tpu as pltpu
from jax.experimental.pallas import tpu_sc as plsc

N = 10000
E = 320000
N_GRAPHS = 64

NC = 2
NS = 16
NW = NC * NS
CHUNK = 100
NCHUNK_W = E // (NW * CHUNK)
NCHUNK_T = E // (NS * CHUNK)
STRIPE = 640
NP = NS * STRIPE


def _make_sc_agg(F):
    Fh = F // 2
    mesh = plsc.VectorSubcoreMesh(core_axis_name="c", subcore_axis_name="s")

    @functools.partial(
        pl.kernel, mesh=mesh,
        compiler_params=pltpu.CompilerParams(use_tc_tiling_on_sc=False),
        out_type=jax.ShapeDtypeStruct((NC, NP, Fh), jnp.float32),
        scratch_types=[
            pltpu.VMEM((NCHUNK_T, CHUNK), jnp.int32),
            pltpu.VMEM((NCHUNK_T, CHUNK), jnp.int32),
            [pltpu.VMEM((CHUNK, Fh), jnp.float32)] * 4,
            pltpu.VMEM((64, Fh), jnp.float32),
            pltpu.VMEM_SHARED((NP, Fh), jnp.float32),
            pltpu.VMEM_SHARED((N, Fh), jnp.float32),
            [pltpu.SemaphoreType.DMA] * 4,
        ],
    )
    def k(g_hbm, e4, out_hbm, src_v, dst_v, bufs, zbuf, acc, g_sp, sems):
        c = lax.axis_index("c")
        s = lax.axis_index("s")

        for i in range(64):
            for j in range(Fh // 16):
                zbuf[i, 16 * j:16 * (j + 1)] = jnp.zeros((16,), jnp.float32)

        def zstripe(kk, carry):
            pltpu.sync_copy(zbuf, acc.at[pl.ds(s * STRIPE + kk * 64, 64)])
            return carry
        lax.fori_loop(0, STRIPE // 64, zstripe, 0)
        pltpu.sync_copy(g_hbm.at[c].at[pl.ds(s * (N // NS), N // NS)],
                        g_sp.at[pl.ds(s * (N // NS), N // NS)])
        plsc.subcore_barrier()

        pltpu.sync_copy(e4.at[0].at[s], src_v)
        pltpu.sync_copy(e4.at[1].at[s], dst_v)

        def gath(j, b):
            pltpu.async_copy(g_sp.at[src_v.at[j]], bufs[b], sems[b])

        def wait(j, b):
            pltpu.make_async_copy(g_sp.at[src_v.at[j]], bufs[b],
                                  sems[b]).wait()

        def scat(j, b):
            pltpu.sync_copy(bufs[b], acc.at[dst_v.at[j]], add=True)

        for b in range(4):
            gath(b, b)

        def quad(t, carry):
            j0 = 4 * t
            for b in range(4):
                wait(j0 + b, b)
                scat(j0 + b, b)
                gath(j0 + b + 4, b)
            return carry
        lax.fori_loop(0, NCHUNK_T // 4 - 1, quad, 0)
        for b in range(4):
            j = NCHUNK_T - 4 + b
            wait(j, b)
            scat(j, b)

        plsc.subcore_barrier()
        pltpu.sync_copy(acc.at[pl.ds(s * STRIPE, STRIPE)],
                        out_hbm.at[c, pl.ds(s * STRIPE, STRIPE)])

    return k


_sc_agg = {f: _make_sc_agg(f) for f in (32, 64)}


def _make_sc_count():
    F = 16
    mesh = plsc.VectorSubcoreMesh(core_axis_name="c", subcore_axis_name="s")

    @functools.partial(
        pl.kernel, mesh=mesh,
        compiler_params=pltpu.CompilerParams(use_tc_tiling_on_sc=False),
        out_type=jax.ShapeDtypeStruct((NC, NP, F), jnp.float32),
        scratch_types=[
            pltpu.VMEM((NCHUNK_W, CHUNK), jnp.int32),
            pltpu.VMEM((CHUNK, F), jnp.float32),
            pltpu.VMEM((64, F), jnp.float32),
            pltpu.VMEM_SHARED((NP, F), jnp.float32),
        ],
    )
    def k(e4, out_hbm, dst_v, ones, zbuf, acc):
        c = lax.axis_index("c")
        s = lax.axis_index("s")

        for i in range(64):
            zbuf[i, 0:16] = jnp.zeros((16,), jnp.float32)
        for i in range(CHUNK):
            ones[i, 0:16] = jnp.ones((16,), jnp.float32)

        def zstripe(kk, carry):
            pltpu.sync_copy(zbuf, acc.at[pl.ds(s * STRIPE + kk * 64, 64)])
            return carry
        lax.fori_loop(0, STRIPE // 64, zstripe, 0)
        plsc.subcore_barrier()

        pltpu.sync_copy(e4.at[1].at[s].at[pl.ds(c * NCHUNK_W, NCHUNK_W)],
                        dst_v)

        def chunk(j, carry):
            pltpu.sync_copy(ones, acc.at[dst_v.at[j]], add=True)
            return carry
        lax.fori_loop(0, NCHUNK_W, chunk, 0)

        plsc.subcore_barrier()
        pltpu.sync_copy(acc.at[pl.ds(s * STRIPE, STRIPE)],
                        out_hbm.at[c, pl.ds(s * STRIPE, STRIPE)])

    return k


_sc_count = _make_sc_count()


def _make_sc_pool():
    F = 64
    mesh = plsc.VectorSubcoreMesh(core_axis_name="c", subcore_axis_name="s")
    NEG = float("-inf")

    @functools.partial(
        pl.kernel, mesh=mesh,
        compiler_params=pltpu.CompilerParams(use_tc_tiling_on_sc=False,
                                             needs_layout_passes=False),
        out_type=jax.ShapeDtypeStruct((N_GRAPHS, F), jnp.float32),
        scratch_types=[
            pltpu.VMEM((N,), jnp.int32),
            pltpu.VMEM((64, F), jnp.float32),
            pltpu.VMEM((2, F), jnp.float32),
        ],
    )
    def k(y_hbm, batch_hbm, out_hbm, b_v, ybuf, obuf):
        c = lax.axis_index("c")
        s = lax.axis_index("s")
        wid = c * NS + s
        g0 = 2 * wid

        pltpu.sync_copy(batch_hbm, b_v)

        def count(i, carry):
            c0, c1, c2 = carry
            one = jnp.ones((16,), jnp.int32)
            zero = jnp.zeros((16,), jnp.int32)
            for u in range(5):
                v = b_v[pl.ds(80 * i + 16 * u, 16)]
                c0 = c0 + jnp.where(v < g0, one, zero)
                c1 = c1 + jnp.where(v < g0 + 1, one, zero)
                c2 = c2 + jnp.where(v < g0 + 2, one, zero)
            return c0, c1, c2
        z = jnp.zeros((16,), jnp.int32)
        c0, c1, c2 = lax.fori_loop(0, N // 80, count, (z, z, z))
        bounds = (jnp.sum(c0), jnp.sum(c1), jnp.sum(c2))

        for gi in range(2):
            lo = bounds[gi]
            hi = bounds[gi + 1]
            n = hi - lo
            minit = tuple(jnp.full((16,), NEG, jnp.float32) for _ in range(4))

            def slab(kk, m):
                pltpu.sync_copy(y_hbm.at[pl.ds(lo + 64 * kk, 64)], ybuf)
                for r in range(64):
                    m = tuple(jnp.maximum(m[q], ybuf[r, 16 * q:16 * (q + 1)])
                              for q in range(4))
                return m
            m = lax.fori_loop(0, n // 64, slab, minit)

            cs = jnp.minimum(jnp.maximum(lo, hi - 64), N - 64)
            pltpu.sync_copy(y_hbm.at[pl.ds(cs, 64)], ybuf)
            for r in range(64):
                ok = jnp.logical_and(cs + r >= lo, cs + r < hi)
                m = tuple(
                    jnp.where(ok,
                              jnp.maximum(m[q], ybuf[r, 16 * q:16 * (q + 1)]),
                              m[q])
                    for q in range(4))

            for q in range(4):
                obuf[gi, 16 * q:16 * (q + 1)] = m[q]

        pltpu.sync_copy(obuf, out_hbm.at[pl.ds(g0, 2)])

    return k


_sc_pool = _make_sc_pool()


def _elu(t):
    return jnp.where(t > 0, t, jnp.exp(t) - 1.0)


GT = 5
BR = N // GT


def _prep_body(cnt_ref, x_ref, w_ref, s_ref, g_ref):
    cnt = cnt_ref[...]
    deg = cnt[0, :, 0:1] + cnt[1, :, 0:1] + 1.0
    sc = jax.lax.rsqrt(deg)
    s_ref[...] = jnp.broadcast_to(sc, s_ref.shape)
    h = jnp.dot(x_ref[...], w_ref[...], preferred_element_type=jnp.float32)
    g = sc * h
    fh = g_ref.shape[2]
    g_ref[0] = g[:, :fh]
    g_ref[1] = g[:, fh:]


def _tc_prep(cnt3, x, W1):
    f = W1.shape[1]
    fh = f // 2
    return pl.pallas_call(
        _prep_body,
        grid=(GT,),
        in_specs=[
            pl.BlockSpec((NC, BR, 16), lambda i: (0, i, 0)),
            pl.BlockSpec((BR, 128), lambda i: (i, 0)),
            pl.BlockSpec((128, f), lambda i: (0, 0)),
        ],
        out_specs=[
            pl.BlockSpec((BR, 8), lambda i: (i, 0)),
            pl.BlockSpec((NC, BR, fh), lambda i: (0, i, 0)),
        ],
        out_shape=[
            jax.ShapeDtypeStruct((NP, 8), jnp.float32),
            jax.ShapeDtypeStruct((NC, N, fh), jnp.float32),
        ],
    )(cnt3, x, W1)


def _layer_body(p_ref, g3_ref, s_ref, w_ref, b_ref, o_ref):
    sc = s_ref[:, 0:1]
    t_lo = sc * (p_ref[0] + g3_ref[0])
    t_hi = sc * (p_ref[1] + g3_ref[1])
    t = jnp.concatenate([t_lo, t_hi], axis=1) + b_ref[...]
    y = _elu(t)
    h = jnp.dot(y, w_ref[...], preferred_element_type=jnp.float32)
    g = sc * h
    fh2 = o_ref.shape[2]
    o_ref[0] = g[:, :fh2]
    o_ref[1] = g[:, fh2:]


def _tc_layer(P3, g3, s, W, b):
    fh = g3.shape[2]
    f2 = W.shape[1]
    fh2 = f2 // 2
    return pl.pallas_call(
        _layer_body,
        grid=(GT,),
        in_specs=[
            pl.BlockSpec((NC, BR, fh), lambda i: (0, i, 0)),
            pl.BlockSpec((NC, BR, fh), lambda i: (0, i, 0)),
            pl.BlockSpec((BR, 8), lambda i: (i, 0)),
            pl.BlockSpec((2 * fh, f2), lambda i: (0, 0)),
            pl.BlockSpec((1, 2 * fh), lambda i: (0, 0)),
        ],
        out_specs=pl.BlockSpec((NC, BR, fh2), lambda i: (0, i, 0)),
        out_shape=jax.ShapeDtypeStruct((NC, N, fh2), jnp.float32),
    )(P3, g3, s, W, b.reshape(1, -1))


def _last_body(p_ref, g3_ref, s_ref, b_ref, o_ref):
    sc = s_ref[:, 0:1]
    t_lo = sc * (p_ref[0] + g3_ref[0])
    t_hi = sc * (p_ref[1] + g3_ref[1])
    t = jnp.concatenate([t_lo, t_hi], axis=1) + b_ref[...]
    o_ref[...] = _elu(t)


def _tc_last(P3, g3, s, b):
    fh = g3.shape[2]
    return pl.pallas_call(
        _last_body,
        grid=(GT,),
        in_specs=[
            pl.BlockSpec((NC, BR, fh), lambda i: (0, i, 0)),
            pl.BlockSpec((NC, BR, fh), lambda i: (0, i, 0)),
            pl.BlockSpec((BR, 8), lambda i: (i, 0)),
            pl.BlockSpec((1, 2 * fh), lambda i: (0, 0)),
        ],
        out_specs=pl.BlockSpec((BR, 2 * fh), lambda i: (i, 0)),
        out_shape=jax.ShapeDtypeStruct((N, 2 * fh), jnp.float32),
    )(P3, g3, s, b.reshape(1, -1))


def _mlp_body(p_ref, w1_ref, b1_ref, w2_ref, b2_ref, w3_ref, b3_ref, o_ref):
    p = p_ref[...]
    p = jnp.where(jnp.isfinite(p), p, 0.0)
    h = _elu(jnp.dot(p, w1_ref[...], preferred_element_type=jnp.float32)
             + b1_ref[...])
    h = _elu(jnp.dot(h, w2_ref[...], preferred_element_type=jnp.float32)
             + b2_ref[...])
    lg = jnp.dot(h, w3_ref[...], preferred_element_type=jnp.float32) \
        + b3_ref[...]
    m = jnp.max(lg, axis=1, keepdims=True)
    lse = m + jnp.log(jnp.sum(jnp.exp(lg - m), axis=1, keepdims=True))
    o_ref[...] = lg - lse


def _tc_mlp(pooled, Wf1, bf1, Wf2, bf2, Wf3, bf3):
    return pl.pallas_call(
        _mlp_body,
        out_shape=jax.ShapeDtypeStruct((N_GRAPHS, 2), jnp.float32),
    )(pooled, Wf1, bf1.reshape(1, -1), Wf2, bf2.reshape(1, -1),
      Wf3, bf3.reshape(1, -1))


def kernel(x, edge_index, batch, W1, b1, W2, b2, Wc0, bc0, Wc1, bc1,
           Wf1, bf1, Wf2, bf2, Wf3, bf3):
    e4 = edge_index.reshape(2, NS, NCHUNK_T, CHUNK)

    cnt3 = _sc_count(e4)
    s, g3 = _tc_prep(cnt3, x, W1)

    def agg(g3_):
        return _sc_agg[2 * g3_.shape[2]](g3_, e4)

    g3 = _tc_layer(agg(g3), g3, s, W2, b1)
    g3 = _tc_layer(agg(g3), g3, s, Wc0, b2)
    g3 = _tc_layer(agg(g3), g3, s, Wc1, bc0)
    y = _tc_last(agg(g3), g3, s, bc1)

    pooled = _sc_pool(y, batch)
    return _tc_mlp(pooled, Wf1, bf1, Wf2, bf2, Wf3, bf3)
